# baseline (speedup 1.0000x reference)
import jax
import jax.numpy as jnp
import numpy as np
from jax import lax
from jax.experimental import pallas as pl
from jax.experimental.pallas import tpu as pltpu
from jax.experimental.pallas import tpu_sc as plsc

N = 10000
E = 320000
D = 128
DE = 16
NG = 64

NC, NS, L = 2, 16, 16
NW = NC * NS
EW = E // NW
CH = 80
IBLK = 2000
CPB = IBLK // CH
NBLK = EW // IBLK
NSLOT = 3
NE = 2
NP = 10240
RPT = NP // NS

RB = 1000
NB = N // RB


def _edge_lin_body(ea_ref, we_ref, be_ref, out_ref):
    r = (
        jnp.dot(ea_ref[...], we_ref[...], preferred_element_type=jnp.float32)
        + be_ref[...]
    ).astype(jnp.bfloat16)
    out_ref[...] = pltpu.bitcast(r, jnp.int32)


def _edge_linear(edge_attr, We, be):
    R = 2000
    return pl.pallas_call(
        _edge_lin_body,
        grid=(E // R,),
        in_specs=[
            pl.BlockSpec((R, DE), lambda i: (i, 0)),
            pl.BlockSpec((DE, D), lambda i: (0, 0)),
            pl.BlockSpec((1, D), lambda i: (0, 0)),
        ],
        out_specs=pl.BlockSpec((R // 2, D), lambda i: (i, 0)),
        out_shape=jax.ShapeDtypeStruct((E // 2, D), jnp.int32),
    )(edge_attr, We, be.reshape(1, D))


def _sc_body(h_hbm, e_hbm, src_hbm, dst_hbm, zero_hbm, out_hbm,
             acc, sidx, didx, rows, ebuf, gsem, esem, ssem):
    c = lax.axis_index("c")
    s = lax.axis_index("s")
    wid = c * NS + s

    pltpu.sync_copy(zero_hbm.at[pl.ds(s * RPT, RPT)], acc.at[pl.ds(s * RPT, RPT)])
    plsc.subcore_barrier()

    def block(b, carry):
        boff = (b * NW + wid) * IBLK
        pltpu.sync_copy(src_hbm.at[pl.ds(boff, IBLK)], sidx)
        pltpu.sync_copy(dst_hbm.at[pl.ds(boff, IBLK)], didx)

        def gath(j, slot):
            return pltpu.make_async_copy(
                h_hbm.at[sidx.at[pl.ds(j * CH, CH)]], rows.at[slot],
                gsem.at[slot])

        def ecp(j, slot):
            off = pl.multiple_of((boff + j * CH) // 2, 8)
            return pltpu.make_async_copy(
                e_hbm.at[pl.ds(off, CH // 2)], ebuf.at[slot],
                esem.at[slot])

        def scat_wait(j, slot):
            pltpu.make_async_copy(rows.at[slot],
                                  acc.at[didx.at[pl.ds(j * CH, CH)]],
                                  ssem.at[slot]).wait()

        gath(0, 0).start()
        ecp(0, 0).start()

        def chunk(j, carry):
            slot = lax.rem(j, NSLOT)
            nslot = lax.rem(j + 1, NSLOT)
            eslot = lax.rem(j, NE)
            neslot = lax.rem(j + 1, NE)

            @pl.when(j + 1 < CPB)
            def _pref():
                @pl.when(j >= 2)
                def _w():
                    scat_wait(j - 2, nslot)
                gath(j + 1, nslot).start()
                ecp(j + 1, neslot).start()

            gath(j, slot).wait()
            ecp(j, eslot).wait()

            for p in range(CH // 2):
                for k in range(D // L):
                    sl = pl.ds(k * L, L)
                    w = ebuf[eslot, p, sl]
                    ea = lax.bitcast_convert_type(
                        jnp.left_shift(w, 16), jnp.float32)
                    eb = lax.bitcast_convert_type(
                        jnp.bitwise_and(w, jnp.int32(-65536)), jnp.float32)
                    rows[slot, 2 * p, sl] = jnp.maximum(
                        rows[slot, 2 * p, sl] + ea, 0.0)
                    rows[slot, 2 * p + 1, sl] = jnp.maximum(
                        rows[slot, 2 * p + 1, sl] + eb, 0.0)

            pltpu.async_copy(rows.at[slot],
                             acc.at[didx.at[pl.ds(j * CH, CH)]],
                             ssem.at[slot], add=True)
            return carry

        lax.fori_loop(0, CPB, chunk, 0)
        for t in range(CPB - 3, CPB):
            scat_wait(t, t % NSLOT)
        return carry

    lax.fori_loop(0, NBLK, block, 0)
    plsc.subcore_barrier()

    pltpu.sync_copy(acc.at[pl.ds(s * RPT, RPT)],
                    out_hbm.at[c, pl.ds(s * RPT, RPT)])


def _sc_aggregate(h, e, src, dst, zeros):
    mesh = plsc.VectorSubcoreMesh(
        core_axis_name="c", subcore_axis_name="s",
        num_cores=NC, num_subcores=NS,
    )
    f = pl.kernel(
        _sc_body,
        out_type=jax.ShapeDtypeStruct((NC, NP, D), jnp.float32),
        mesh=mesh,
        scratch_types=[
            pltpu.VMEM_SHARED((NP, D), jnp.float32),
            pltpu.VMEM((IBLK,), jnp.int32),
            pltpu.VMEM((IBLK,), jnp.int32),
            pltpu.VMEM((NSLOT, CH, D), jnp.float32),
            pltpu.VMEM((NE, CH // 2, D), jnp.int32),
            pltpu.SemaphoreType.DMA((NSLOT,)),
            pltpu.SemaphoreType.DMA((NE,)),
            pltpu.SemaphoreType.DMA((NSLOT,)),
        ],
    )
    return f(h, e, src, dst, zeros)


def _node_core(p0_ref, p1_ref, h_ref, w1_ref, b1_ref, w2_ref, b2_ref,
               sc_ref, g_ref, beta_ref):
    z = sc_ref[...] * h_ref[...] + p0_ref[...] + p1_ref[...]
    a = jnp.maximum(
        jnp.dot(z, w1_ref[...], preferred_element_type=jnp.float32) + b1_ref[...],
        0.0,
    )
    z2 = jnp.dot(a, w2_ref[...], preferred_element_type=jnp.float32) + b2_ref[...]
    mu = jnp.mean(z2, axis=1, keepdims=True)
    d = z2 - mu
    var = jnp.mean(d * d, axis=1, keepdims=True)
    zn = d * lax.rsqrt(var + 1e-5) * g_ref[...] + beta_ref[...]
    return jnp.maximum(zn, 0.0)


def _node_body(p0_ref, p1_ref, h_ref, w1_ref, b1_ref, w2_ref, b2_ref,
               sc_ref, g_ref, beta_ref, out_ref):
    out_ref[...] = _node_core(p0_ref, p1_ref, h_ref, w1_ref, b1_ref, w2_ref,
                              b2_ref, sc_ref, g_ref, beta_ref)


def _node_pool_body(p0_ref, p1_ref, h_ref, w1_ref, b1_ref, w2_ref, b2_ref,
                    sc_ref, g_ref, beta_ref, batch_ref, out_ref, ge_ref,
                    sums_ref, cnt_ref):
    i = pl.program_id(0)
    hout = _node_core(p0_ref, p1_ref, h_ref, w1_ref, b1_ref, w2_ref, b2_ref,
                      sc_ref, g_ref, beta_ref)
    out_ref[...] = hout

    @pl.when(i == 0)
    def _init():
        sums_ref[...] = jnp.zeros((NG, D), jnp.float32)
        cnt_ref[...] = jnp.zeros((NG, D), jnp.float32)

    b = batch_ref[0, 0, :]
    oh = (b[:, None]
          == lax.broadcasted_iota(jnp.int32, (RB, NG), 1)).astype(jnp.float32)
    dn = (((0,), (0,)), ((), ()))
    sums_ref[...] += lax.dot_general(oh, hout, dn,
                                     preferred_element_type=jnp.float32)
    cnt_ref[...] += lax.dot_general(oh, jnp.ones((RB, D), jnp.float32), dn,
                                    preferred_element_type=jnp.float32)

    @pl.when(i == NB - 1)
    def _fin():
        ge_ref[...] = sums_ref[...] / jnp.maximum(cnt_ref[...], 1.0)


def _node_update(p0, p1, h, W1, b1, W2, b2, scale, g, beta):
    row = pl.BlockSpec((1, D), lambda i: (0, 0))
    return pl.pallas_call(
        _node_body,
        grid=(NB,),
        in_specs=[
            pl.BlockSpec((RB, D), lambda i: (i, 0)),
            pl.BlockSpec((RB, D), lambda i: (i, 0)),
            pl.BlockSpec((RB, D), lambda i: (i, 0)),
            pl.BlockSpec((D, D), lambda i: (0, 0)),
            row,
            pl.BlockSpec((D, D), lambda i: (0, 0)),
            row,
            pl.BlockSpec((1, 1), lambda i: (0, 0)),
            row,
            row,
        ],
        out_specs=pl.BlockSpec((RB, D), lambda i: (i, 0)),
        out_shape=jax.ShapeDtypeStruct((N, D), jnp.float32),
    )(p0, p1, h, W1, b1.reshape(1, D), W2, b2.reshape(1, D),
      scale, g.reshape(1, D), beta.reshape(1, D))


def _node_update_pool(p0, p1, h, W1, b1, W2, b2, scale, g, beta, batch3):
    row = pl.BlockSpec((1, D), lambda i: (0, 0))
    return pl.pallas_call(
        _node_pool_body,
        grid=(NB,),
        in_specs=[
            pl.BlockSpec((RB, D), lambda i: (i, 0)),
            pl.BlockSpec((RB, D), lambda i: (i, 0)),
            pl.BlockSpec((RB, D), lambda i: (i, 0)),
            pl.BlockSpec((D, D), lambda i: (0, 0)),
            row,
            pl.BlockSpec((D, D), lambda i: (0, 0)),
            row,
            pl.BlockSpec((1, 1), lambda i: (0, 0)),
            row,
            row,
            pl.BlockSpec((1, 1, RB), lambda i: (i, 0, 0)),
        ],
        out_specs=[
            pl.BlockSpec((RB, D), lambda i: (i, 0)),
            pl.BlockSpec((NG, D), lambda i: (0, 0)),
        ],
        out_shape=[
            jax.ShapeDtypeStruct((N, D), jnp.float32),
            jax.ShapeDtypeStruct((NG, D), jnp.float32),
        ],
        scratch_shapes=[
            pltpu.VMEM((NG, D), jnp.float32),
            pltpu.VMEM((NG, D), jnp.float32),
        ],
    )(p0, p1, h, W1, b1.reshape(1, D), W2, b2.reshape(1, D),
      scale, g.reshape(1, D), beta.reshape(1, D), batch3)


def kernel(x, edge_index, edge_attr, batch,
           We0, be0, W1_0, b1_0, W2_0, b2_0, eps0, g0, beta0,
           We1, be1, W1_1, b1_1, W2_1, b2_1, eps1, g1, beta1):
    src = edge_index[0]
    dst = edge_index[1]
    zeros = jnp.zeros((NP, D), jnp.float32)
    batch3 = batch.reshape(NB, 1, RB)

    e0 = _edge_linear(edge_attr, We0, be0)
    e1 = _edge_linear(edge_attr, We1, be1)

    p = _sc_aggregate(x, e0, src, dst, zeros)[:, :N]
    h = _node_update(p[0], p[1], x, W1_0, b1_0, W2_0, b2_0,
                     (1.0 + eps0).reshape(1, 1), g0, beta0)

    p = _sc_aggregate(h, e1, src, dst, zeros)[:, :N]
    h, ge = _node_update_pool(p[0], p[1], h, W1_1, b1_1, W2_1, b2_1,
                              (1.0 + eps1).reshape(1, 1), g1, beta1, batch3)
    return ge, h


# speedup vs baseline: 2.3002x; 2.3002x over previous
import jax
import jax.numpy as jnp
import numpy as np
from jax import lax
from jax.experimental import pallas as pl
from jax.experimental.pallas import tpu as pltpu
from jax.experimental.pallas import tpu_sc as plsc

N = 10000
E = 320000
D = 128
DE = 16
NG = 64

NC, NS, L = 2, 16, 16
NW = NC * NS
EW = E // NW
CH = 80
IBLK = 2000
CPB = IBLK // CH
NBLK = EW // IBLK
NSLOT = 3
NE = 2
NP = 10240
RPT = NP // NS

RB = 1000
NB = N // RB


def _edge_lin_body(ea_ref, we_ref, be_ref, out_ref):
    r = (
        jnp.dot(ea_ref[...], we_ref[...], preferred_element_type=jnp.float32)
        + be_ref[...]
    ).astype(jnp.bfloat16)
    out_ref[...] = pltpu.bitcast(r, jnp.int32)


def _edge_linear(edge_attr, We, be):
    R = 2000
    return pl.pallas_call(
        _edge_lin_body,
        grid=(E // R,),
        in_specs=[
            pl.BlockSpec((R, DE), lambda i: (i, 0)),
            pl.BlockSpec((DE, D), lambda i: (0, 0)),
            pl.BlockSpec((1, D), lambda i: (0, 0)),
        ],
        out_specs=pl.BlockSpec((R // 2, D), lambda i: (i, 0)),
        out_shape=jax.ShapeDtypeStruct((E // 2, D), jnp.int32),
    )(edge_attr, We, be.reshape(1, D))


def _sc_body(h_hbm, e_hbm, src_hbm, dst_hbm, zero_hbm, out_hbm,
             acc, sidx, didx, rows, ebuf, gsem, esem, ssem):
    c = lax.axis_index("c")
    s = lax.axis_index("s")
    wid = c * NS + s

    pltpu.sync_copy(zero_hbm.at[pl.ds(s * RPT, RPT)], acc.at[pl.ds(s * RPT, RPT)])
    plsc.subcore_barrier()

    def block(b, carry):
        boff = (b * NW + wid) * IBLK
        pltpu.sync_copy(src_hbm.at[pl.ds(boff, IBLK)], sidx)
        pltpu.sync_copy(dst_hbm.at[pl.ds(boff, IBLK)], didx)

        def gath(j, slot):
            return pltpu.make_async_copy(
                h_hbm.at[sidx.at[pl.ds(j * CH, CH)]], rows.at[slot],
                gsem.at[slot])

        def ecp(j, slot):
            off = pl.multiple_of((boff + j * CH) // 2, 8)
            return pltpu.make_async_copy(
                e_hbm.at[pl.ds(off, CH // 2)], ebuf.at[slot],
                esem.at[slot])

        def scat_wait(j, slot):
            pltpu.make_async_copy(rows.at[slot],
                                  acc.at[didx.at[pl.ds(j * CH, CH)]],
                                  ssem.at[slot]).wait()

        gath(0, 0).start()
        ecp(0, 0).start()

        def chunk(j, carry):
            slot = lax.rem(j, NSLOT)
            nslot = lax.rem(j + 1, NSLOT)
            eslot = lax.rem(j, NE)
            neslot = lax.rem(j + 1, NE)

            @pl.when(j + 1 < CPB)
            def _pref():
                @pl.when(j >= 2)
                def _w():
                    scat_wait(j - 2, nslot)
                gath(j + 1, nslot).start()
                ecp(j + 1, neslot).start()

            gath(j, slot).wait()
            ecp(j, eslot).wait()

            @plsc.parallel_loop(0, CH // 2, unroll=4)
            def pair(p):
                for k in range(D // L):
                    sl = pl.ds(k * L, L)
                    w = ebuf[eslot, p, sl]
                    ea = lax.bitcast_convert_type(
                        jnp.left_shift(w, 16), jnp.float32)
                    eb = lax.bitcast_convert_type(
                        jnp.bitwise_and(w, jnp.int32(-65536)), jnp.float32)
                    rows[slot, 2 * p, sl] = jnp.maximum(
                        rows[slot, 2 * p, sl] + ea, 0.0)
                    rows[slot, 2 * p + 1, sl] = jnp.maximum(
                        rows[slot, 2 * p + 1, sl] + eb, 0.0)

            pltpu.async_copy(rows.at[slot],
                             acc.at[didx.at[pl.ds(j * CH, CH)]],
                             ssem.at[slot], add=True)
            return carry

        lax.fori_loop(0, CPB, chunk, 0)
        for t in range(CPB - 3, CPB):
            scat_wait(t, t % NSLOT)
        return carry

    lax.fori_loop(0, NBLK, block, 0)
    plsc.subcore_barrier()

    pltpu.sync_copy(acc.at[pl.ds(s * RPT, RPT)],
                    out_hbm.at[c, pl.ds(s * RPT, RPT)])


def _sc_aggregate(h, e, src, dst, zeros):
    mesh = plsc.VectorSubcoreMesh(
        core_axis_name="c", subcore_axis_name="s",
        num_cores=NC, num_subcores=NS,
    )
    f = pl.kernel(
        _sc_body,
        out_type=jax.ShapeDtypeStruct((NC, NP, D), jnp.float32),
        mesh=mesh,
        scratch_types=[
            pltpu.VMEM_SHARED((NP, D), jnp.float32),
            pltpu.VMEM((IBLK,), jnp.int32),
            pltpu.VMEM((IBLK,), jnp.int32),
            pltpu.VMEM((NSLOT, CH, D), jnp.float32),
            pltpu.VMEM((NE, CH // 2, D), jnp.int32),
            pltpu.SemaphoreType.DMA((NSLOT,)),
            pltpu.SemaphoreType.DMA((NE,)),
            pltpu.SemaphoreType.DMA((NSLOT,)),
        ],
    )
    return f(h, e, src, dst, zeros)


def _node_core(p0_ref, p1_ref, h_ref, w1_ref, b1_ref, w2_ref, b2_ref,
               sc_ref, g_ref, beta_ref):
    z = sc_ref[...] * h_ref[...] + p0_ref[...] + p1_ref[...]
    a = jnp.maximum(
        jnp.dot(z, w1_ref[...], preferred_element_type=jnp.float32) + b1_ref[...],
        0.0,
    )
    z2 = jnp.dot(a, w2_ref[...], preferred_element_type=jnp.float32) + b2_ref[...]
    mu = jnp.mean(z2, axis=1, keepdims=True)
    d = z2 - mu
    var = jnp.mean(d * d, axis=1, keepdims=True)
    zn = d * lax.rsqrt(var + 1e-5) * g_ref[...] + beta_ref[...]
    return jnp.maximum(zn, 0.0)


def _node_body(p0_ref, p1_ref, h_ref, w1_ref, b1_ref, w2_ref, b2_ref,
               sc_ref, g_ref, beta_ref, out_ref):
    out_ref[...] = _node_core(p0_ref, p1_ref, h_ref, w1_ref, b1_ref, w2_ref,
                              b2_ref, sc_ref, g_ref, beta_ref)


def _node_pool_body(p0_ref, p1_ref, h_ref, w1_ref, b1_ref, w2_ref, b2_ref,
                    sc_ref, g_ref, beta_ref, batch_ref, out_ref, ge_ref,
                    sums_ref, cnt_ref):
    i = pl.program_id(0)
    hout = _node_core(p0_ref, p1_ref, h_ref, w1_ref, b1_ref, w2_ref, b2_ref,
                      sc_ref, g_ref, beta_ref)
    out_ref[...] = hout

    @pl.when(i == 0)
    def _init():
        sums_ref[...] = jnp.zeros((NG, D), jnp.float32)
        cnt_ref[...] = jnp.zeros((NG, D), jnp.float32)

    b = batch_ref[0, 0, :]
    oh = (b[:, None]
          == lax.broadcasted_iota(jnp.int32, (RB, NG), 1)).astype(jnp.float32)
    dn = (((0,), (0,)), ((), ()))
    sums_ref[...] += lax.dot_general(oh, hout, dn,
                                     preferred_element_type=jnp.float32)
    cnt_ref[...] += lax.dot_general(oh, jnp.ones((RB, D), jnp.float32), dn,
                                    preferred_element_type=jnp.float32)

    @pl.when(i == NB - 1)
    def _fin():
        ge_ref[...] = sums_ref[...] / jnp.maximum(cnt_ref[...], 1.0)


def _node_update(p0, p1, h, W1, b1, W2, b2, scale, g, beta):
    row = pl.BlockSpec((1, D), lambda i: (0, 0))
    return pl.pallas_call(
        _node_body,
        grid=(NB,),
        in_specs=[
            pl.BlockSpec((RB, D), lambda i: (i, 0)),
            pl.BlockSpec((RB, D), lambda i: (i, 0)),
            pl.BlockSpec((RB, D), lambda i: (i, 0)),
            pl.BlockSpec((D, D), lambda i: (0, 0)),
            row,
            pl.BlockSpec((D, D), lambda i: (0, 0)),
            row,
            pl.BlockSpec((1, 1), lambda i: (0, 0)),
            row,
            row,
        ],
        out_specs=pl.BlockSpec((RB, D), lambda i: (i, 0)),
        out_shape=jax.ShapeDtypeStruct((N, D), jnp.float32),
    )(p0, p1, h, W1, b1.reshape(1, D), W2, b2.reshape(1, D),
      scale, g.reshape(1, D), beta.reshape(1, D))


def _node_update_pool(p0, p1, h, W1, b1, W2, b2, scale, g, beta, batch3):
    row = pl.BlockSpec((1, D), lambda i: (0, 0))
    return pl.pallas_call(
        _node_pool_body,
        grid=(NB,),
        in_specs=[
            pl.BlockSpec((RB, D), lambda i: (i, 0)),
            pl.BlockSpec((RB, D), lambda i: (i, 0)),
            pl.BlockSpec((RB, D), lambda i: (i, 0)),
            pl.BlockSpec((D, D), lambda i: (0, 0)),
            row,
            pl.BlockSpec((D, D), lambda i: (0, 0)),
            row,
            pl.BlockSpec((1, 1), lambda i: (0, 0)),
            row,
            row,
            pl.BlockSpec((1, 1, RB), lambda i: (i, 0, 0)),
        ],
        out_specs=[
            pl.BlockSpec((RB, D), lambda i: (i, 0)),
            pl.BlockSpec((NG, D), lambda i: (0, 0)),
        ],
        out_shape=[
            jax.ShapeDtypeStruct((N, D), jnp.float32),
            jax.ShapeDtypeStruct((NG, D), jnp.float32),
        ],
        scratch_shapes=[
            pltpu.VMEM((NG, D), jnp.float32),
            pltpu.VMEM((NG, D), jnp.float32),
        ],
    )(p0, p1, h, W1, b1.reshape(1, D), W2, b2.reshape(1, D),
      scale, g.reshape(1, D), beta.reshape(1, D), batch3)


def kernel(x, edge_index, edge_attr, batch,
           We0, be0, W1_0, b1_0, W2_0, b2_0, eps0, g0, beta0,
           We1, be1, W1_1, b1_1, W2_1, b2_1, eps1, g1, beta1):
    src = edge_index[0]
    dst = edge_index[1]
    zeros = jnp.zeros((NP, D), jnp.float32)
    batch3 = batch.reshape(NB, 1, RB)

    e0 = _edge_linear(edge_attr, We0, be0)
    e1 = _edge_linear(edge_attr, We1, be1)

    p = _sc_aggregate(x, e0, src, dst, zeros)[:, :N]
    h = _node_update(p[0], p[1], x, W1_0, b1_0, W2_0, b2_0,
                     (1.0 + eps0).reshape(1, 1), g0, beta0)

    p = _sc_aggregate(h, e1, src, dst, zeros)[:, :N]
    h, ge = _node_update_pool(p[0], p[1], h, W1_1, b1_1, W2_1, b2_1,
                              (1.0 + eps1).reshape(1, 1), g1, beta1, batch3)
    return ge, h


# speedup vs baseline: 2.3584x; 1.0253x over previous
import jax
import jax.numpy as jnp
import numpy as np
from jax import lax
from jax.experimental import pallas as pl
from jax.experimental.pallas import tpu as pltpu
from jax.experimental.pallas import tpu_sc as plsc

N = 10000
E = 320000
D = 128
DE = 16
NG = 64

NC, NS, L = 2, 16, 16
NW = NC * NS
EW = E // NW
CH = 80
IBLK = 2000
CPB = IBLK // CH
NBLK = EW // IBLK
NSLOT = 3
NE = 2
NP = 10240
RPT = NP // NS

RB = 1000
NB = N // RB


def _edge_lin_body(ea_ref, we_ref, be_ref, out_ref):
    r = (
        jnp.dot(ea_ref[...], we_ref[...], preferred_element_type=jnp.float32)
        + be_ref[...]
    ).astype(jnp.bfloat16)
    out_ref[...] = pltpu.bitcast(r, jnp.int32)


def _edge_linear(edge_attr, We, be):
    R = 2000
    return pl.pallas_call(
        _edge_lin_body,
        grid=(E // R,),
        in_specs=[
            pl.BlockSpec((R, DE), lambda i: (i, 0)),
            pl.BlockSpec((DE, D), lambda i: (0, 0)),
            pl.BlockSpec((1, D), lambda i: (0, 0)),
        ],
        out_specs=pl.BlockSpec((R // 2, D), lambda i: (i, 0)),
        out_shape=jax.ShapeDtypeStruct((E // 2, D), jnp.int32),
    )(edge_attr, We, be.reshape(1, D))


def _sc_body(h_hbm, e_hbm, src_hbm, dst_hbm, zero_hbm, out_hbm,
             acc, sidx, didx, rows, ebuf, gsem, esem, ssem):
    c = lax.axis_index("c")
    s = lax.axis_index("s")
    wid = c * NS + s

    pltpu.sync_copy(zero_hbm.at[pl.ds(s * RPT, RPT)], acc.at[pl.ds(s * RPT, RPT)])
    plsc.subcore_barrier()

    def block(b, carry):
        boff = (b * NW + wid) * IBLK
        pltpu.sync_copy(src_hbm.at[pl.ds(boff, IBLK)], sidx)
        pltpu.sync_copy(dst_hbm.at[pl.ds(boff, IBLK)], didx)

        def gath(j, slot):
            return pltpu.make_async_copy(
                h_hbm.at[sidx.at[pl.ds(j * CH, CH)]], rows.at[slot],
                gsem.at[slot])

        def ecp(j, slot):
            off = pl.multiple_of((boff + j * CH) // 2, 8)
            return pltpu.make_async_copy(
                e_hbm.at[pl.ds(off, CH // 2)], ebuf.at[slot],
                esem.at[slot])

        def scat_wait(j, slot):
            pltpu.make_async_copy(rows.at[slot],
                                  acc.at[didx.at[pl.ds(j * CH, CH)]],
                                  ssem.at[slot]).wait()

        gath(0, 0).start()
        ecp(0, 0).start()

        def chunk(j, carry):
            slot = lax.rem(j, NSLOT)
            nslot = lax.rem(j + 1, NSLOT)
            eslot = lax.rem(j, NE)
            neslot = lax.rem(j + 1, NE)

            @pl.when(j + 1 < CPB)
            def _pref():
                @pl.when(j >= 2)
                def _w():
                    scat_wait(j - 2, nslot)
                gath(j + 1, nslot).start()
                ecp(j + 1, neslot).start()

            gath(j, slot).wait()
            ecp(j, eslot).wait()

            @plsc.parallel_loop(0, CH // 2, unroll=4)
            def pair(p):
                for k in range(D // L):
                    sl = pl.ds(k * L, L)
                    w = ebuf[eslot, p, sl]
                    ea = lax.bitcast_convert_type(
                        jnp.left_shift(w, 16), jnp.float32)
                    eb = lax.bitcast_convert_type(
                        jnp.bitwise_and(w, jnp.int32(-65536)), jnp.float32)
                    rows[slot, 2 * p, sl] = jnp.maximum(
                        rows[slot, 2 * p, sl] + ea, 0.0)
                    rows[slot, 2 * p + 1, sl] = jnp.maximum(
                        rows[slot, 2 * p + 1, sl] + eb, 0.0)

            pltpu.async_copy(rows.at[slot],
                             acc.at[didx.at[pl.ds(j * CH, CH)]],
                             ssem.at[slot], add=True)
            return carry

        lax.fori_loop(0, CPB, chunk, 0)
        for t in range(CPB - 3, CPB):
            scat_wait(t, t % NSLOT)
        return carry

    lax.fori_loop(0, NBLK, block, 0)
    plsc.subcore_barrier()

    pltpu.sync_copy(acc.at[pl.ds(s * RPT, RPT)],
                    out_hbm.at[c, pl.ds(s * RPT, RPT)])


def _sc_aggregate(h, e, src, dst, zeros):
    mesh = plsc.VectorSubcoreMesh(
        core_axis_name="c", subcore_axis_name="s",
        num_cores=NC, num_subcores=NS,
    )
    f = pl.kernel(
        _sc_body,
        out_type=jax.ShapeDtypeStruct((NC, NP, D), jnp.float32),
        mesh=mesh,
        scratch_types=[
            pltpu.VMEM_SHARED((NP, D), jnp.float32),
            pltpu.VMEM((IBLK,), jnp.int32),
            pltpu.VMEM((IBLK,), jnp.int32),
            pltpu.VMEM((NSLOT, CH, D), jnp.float32),
            pltpu.VMEM((NE, CH // 2, D), jnp.int32),
            pltpu.SemaphoreType.DMA((NSLOT,)),
            pltpu.SemaphoreType.DMA((NE,)),
            pltpu.SemaphoreType.DMA((NSLOT,)),
        ],
    )
    return f(h, e, src, dst, zeros)


def _node_core(p0_ref, p1_ref, h_ref, w1_ref, b1_ref, w2_ref, b2_ref,
               sc_ref, g_ref, beta_ref):
    z = sc_ref[...] * h_ref[...] + p0_ref[0] + p1_ref[0]
    a = jnp.maximum(
        jnp.dot(z, w1_ref[...], preferred_element_type=jnp.float32) + b1_ref[...],
        0.0,
    )
    z2 = jnp.dot(a, w2_ref[...], preferred_element_type=jnp.float32) + b2_ref[...]
    mu = jnp.mean(z2, axis=1, keepdims=True)
    d = z2 - mu
    var = jnp.mean(d * d, axis=1, keepdims=True)
    zn = d * lax.rsqrt(var + 1e-5) * g_ref[...] + beta_ref[...]
    return jnp.maximum(zn, 0.0)


def _node_body(p0_ref, p1_ref, h_ref, w1_ref, b1_ref, w2_ref, b2_ref,
               sc_ref, g_ref, beta_ref, out_ref):
    out_ref[...] = _node_core(p0_ref, p1_ref, h_ref, w1_ref, b1_ref, w2_ref,
                              b2_ref, sc_ref, g_ref, beta_ref)


def _node_pool_body(p0_ref, p1_ref, h_ref, w1_ref, b1_ref, w2_ref, b2_ref,
                    sc_ref, g_ref, beta_ref, batch_ref, out_ref, ge_ref,
                    sums_ref, cnt_ref):
    i = pl.program_id(0)
    hout = _node_core(p0_ref, p1_ref, h_ref, w1_ref, b1_ref, w2_ref, b2_ref,
                      sc_ref, g_ref, beta_ref)
    out_ref[...] = hout

    @pl.when(i == 0)
    def _init():
        sums_ref[...] = jnp.zeros((NG, D), jnp.float32)
        cnt_ref[...] = jnp.zeros((NG, D), jnp.float32)

    b = batch_ref[0, 0, :]
    oh = (b[:, None]
          == lax.broadcasted_iota(jnp.int32, (RB, NG), 1)).astype(jnp.float32)
    dn = (((0,), (0,)), ((), ()))
    sums_ref[...] += lax.dot_general(oh, hout, dn,
                                     preferred_element_type=jnp.float32)
    cnt_ref[...] += lax.dot_general(oh, jnp.ones((RB, D), jnp.float32), dn,
                                    preferred_element_type=jnp.float32)

    @pl.when(i == NB - 1)
    def _fin():
        ge_ref[...] = sums_ref[...] / jnp.maximum(cnt_ref[...], 1.0)


def _node_update(p, h, W1, b1, W2, b2, scale, g, beta):
    row = pl.BlockSpec((1, D), lambda i: (0, 0))
    return pl.pallas_call(
        _node_body,
        grid=(NB,),
        in_specs=[
            pl.BlockSpec((1, RB, D), lambda i: (0, i, 0)),
            pl.BlockSpec((1, RB, D), lambda i: (1, i, 0)),
            pl.BlockSpec((RB, D), lambda i: (i, 0)),
            pl.BlockSpec((D, D), lambda i: (0, 0)),
            row,
            pl.BlockSpec((D, D), lambda i: (0, 0)),
            row,
            pl.BlockSpec((1, 1), lambda i: (0, 0)),
            row,
            row,
        ],
        out_specs=pl.BlockSpec((RB, D), lambda i: (i, 0)),
        out_shape=jax.ShapeDtypeStruct((N, D), jnp.float32),
    )(p, p, h, W1, b1.reshape(1, D), W2, b2.reshape(1, D),
      scale, g.reshape(1, D), beta.reshape(1, D))


def _node_update_pool(p, h, W1, b1, W2, b2, scale, g, beta, batch3):
    row = pl.BlockSpec((1, D), lambda i: (0, 0))
    return pl.pallas_call(
        _node_pool_body,
        grid=(NB,),
        in_specs=[
            pl.BlockSpec((1, RB, D), lambda i: (0, i, 0)),
            pl.BlockSpec((1, RB, D), lambda i: (1, i, 0)),
            pl.BlockSpec((RB, D), lambda i: (i, 0)),
            pl.BlockSpec((D, D), lambda i: (0, 0)),
            row,
            pl.BlockSpec((D, D), lambda i: (0, 0)),
            row,
            pl.BlockSpec((1, 1), lambda i: (0, 0)),
            row,
            row,
            pl.BlockSpec((1, 1, RB), lambda i: (i, 0, 0)),
        ],
        out_specs=[
            pl.BlockSpec((RB, D), lambda i: (i, 0)),
            pl.BlockSpec((NG, D), lambda i: (0, 0)),
        ],
        out_shape=[
            jax.ShapeDtypeStruct((N, D), jnp.float32),
            jax.ShapeDtypeStruct((NG, D), jnp.float32),
        ],
        scratch_shapes=[
            pltpu.VMEM((NG, D), jnp.float32),
            pltpu.VMEM((NG, D), jnp.float32),
        ],
    )(p, p, h, W1, b1.reshape(1, D), W2, b2.reshape(1, D),
      scale, g.reshape(1, D), beta.reshape(1, D), batch3)


def kernel(x, edge_index, edge_attr, batch,
           We0, be0, W1_0, b1_0, W2_0, b2_0, eps0, g0, beta0,
           We1, be1, W1_1, b1_1, W2_1, b2_1, eps1, g1, beta1):
    src = edge_index[0]
    dst = edge_index[1]
    zeros = jnp.zeros((NP, D), jnp.float32)
    batch3 = batch.reshape(NB, 1, RB)

    e0 = _edge_linear(edge_attr, We0, be0)
    e1 = _edge_linear(edge_attr, We1, be1)

    p = _sc_aggregate(x, e0, src, dst, zeros)
    h = _node_update(p, x, W1_0, b1_0, W2_0, b2_0,
                     (1.0 + eps0).reshape(1, 1), g0, beta0)

    p = _sc_aggregate(h, e1, src, dst, zeros)
    h, ge = _node_update_pool(p, h, W1_1, b1_1, W2_1, b2_1,
                              (1.0 + eps1).reshape(1, 1), g1, beta1, batch3)
    return ge, h
